# baseline (speedup 1.0000x reference)
import functools

import jax
import jax.numpy as jnp
from jax import lax
from jax.experimental import pallas as pl
from jax.experimental.pallas import tpu as pltpu
from jax.experimental.pallas import tpu_sc as plsc

E = 320000
NN = 10000
D = 128
DE = 16

NC = 2
NS = 16
NW = NC * NS

GATE_BLK = 2048
SC_SPLITS = (115200, 102400, 102400)
SC_BASES = (0, 115200, 217600)
GATE_OFF = (0, 56, 106)
GATE_GRID = (57, 51, 51)
W_OFF = (0, 512, 512)

RPT = 624
RPT_LAST = 640
ZR = 80

ADD_BLK = 2000


def _gate_body(xj_ref, ei_ref, xi_ref, w1_ref, w2_ref, w3_ref, b_ref, out_ref):
    dn = (((1,), (1,)), ((), ()))
    s = jax.lax.dot_general(w1_ref[...], xj_ref[...], dn,
                            preferred_element_type=jnp.float32)
    s = s + jax.lax.dot_general(w2_ref[...], ei_ref[...], dn,
                                preferred_element_type=jnp.float32)
    s = s + jax.lax.dot_general(w3_ref[...], xi_ref[...], dn,
                                preferred_element_type=jnp.float32)
    out_ref[...] = jnp.tanh(s + b_ref[0])[0]


def _make_gate_call(off, grid):
    return pl.pallas_call(
        _gate_body,
        grid=(grid,),
        in_specs=[
            pl.BlockSpec((GATE_BLK, D), lambda i: (i + off, 0)),
            pl.BlockSpec((GATE_BLK, DE), lambda i: (i + off, 0)),
            pl.BlockSpec((GATE_BLK, D), lambda i: (i + off, 0)),
            pl.BlockSpec((1, D), lambda i: (0, 0)),
            pl.BlockSpec((1, DE), lambda i: (0, 0)),
            pl.BlockSpec((1, D), lambda i: (0, 0)),
            pl.BlockSpec((1,), lambda i: (0,)),
        ],
        out_specs=pl.BlockSpec((GATE_BLK,), lambda i: (i,)),
        out_shape=jax.ShapeDtypeStruct((grid * GATE_BLK,), jnp.float32),
    )


_gate_calls = [_make_gate_call(o, g) for o, g in zip(GATE_OFF, GATE_GRID)]


_mesh = plsc.VectorSubcoreMesh(core_axis_name="c", subcore_axis_name="s")


def _make_sc_scatter(e_base, epw, CH, w_off):
    nchunk = epw // CH
    assert nchunk * CH == epw and epw % 8 == 0 and CH % 8 == 0 and CH <= 128
    assert CH % 2 == 0 and w_off % 8 == 0

    @functools.partial(
        pl.kernel,
        mesh=_mesh,
        out_type=jax.ShapeDtypeStruct((NC * NN, D), jnp.float32),
        scratch_types=[
            pltpu.VMEM((2, CH, D), jnp.float32),
            pltpu.VMEM((2, CH), jnp.float32),
            pltpu.VMEM((2, CH), jnp.int32),
            pltpu.VMEM((ZR, D), jnp.float32),
            pltpu.VMEM_SHARED((NN, D), jnp.float32),
            pltpu.SemaphoreType.DMA,
            pltpu.SemaphoreType.DMA,
        ],
        compiler_params=pltpu.CompilerParams(needs_layout_passes=False),
    )
    def _sc_scatter(msg_hbm, w_hbm, idx_hbm, out_hbm, msg_v, w_v, idx_v, z_v,
                    acc, isem0, isem1):
        cid = lax.axis_index("c")
        sid = lax.axis_index("s")
        wid = cid * NS + sid
        rbase = w_off + wid * epw
        abase = e_base + wid * epw
        isems = (isem0, isem1)

        def _zrow(r, carry):
            for c in range(D // 16):
                z_v[r, pl.ds(c * 16, 16)] = jnp.zeros((16,), jnp.float32)
            return carry

        lax.fori_loop(0, ZR, _zrow, 0)

        def _zcopy(k, carry):
            pltpu.sync_copy(z_v, acc.at[pl.ds(sid * RPT + k * ZR, ZR)])
            return carry

        lax.fori_loop(0, RPT_LAST // ZR, _zcopy, 0)
        plsc.subcore_barrier()

        def _in_dmas(j, b):
            return (
                pltpu.make_async_copy(
                    msg_hbm.at[pl.ds(abase + j * CH, CH)], msg_v.at[b],
                    isems[b]),
                pltpu.make_async_copy(
                    w_hbm.at[pl.ds(rbase + j * CH, CH)], w_v.at[b],
                    isems[b]),
                pltpu.make_async_copy(
                    idx_hbm.at[pl.ds(abase + j * CH, CH)], idx_v.at[b],
                    isems[b]),
            )

        def _start_in(j, b):
            for d in _in_dmas(j, b):
                d.start()

        def _process(j, b):
            for d in _in_dmas(j, b):
                d.wait()

            def _mrow(r2, inner):
                for u in range(2):
                    r = 2 * r2 + u
                    wb = plsc.load_gather(w_v.at[b],
                                          [jnp.full((16,), r, jnp.int32)])
                    for c in range(D // 16):
                        sl = pl.ds(c * 16, 16)
                        msg_v[b, r, sl] = msg_v[b, r, sl] * wb
                return inner

            lax.fori_loop(0, CH // 2, _mrow, 0)
            pltpu.sync_copy(msg_v.at[b], acc.at[idx_v.at[b]], add=True)

        _start_in(0, 0)

        def _pair(k, carry):
            j0 = 2 * k
            _start_in(j0 + 1, 1)
            _process(j0, 0)
            _start_in(j0 + 2, 0)
            _process(j0 + 1, 1)
            return carry

        if nchunk % 2 == 1:
            lax.fori_loop(0, (nchunk - 1) // 2, _pair, 0)
            _process(nchunk - 1, 0)
        else:
            lax.fori_loop(0, (nchunk - 2) // 2, _pair, 0)
            _start_in(nchunk - 1, 1)
            _process(nchunk - 2, 0)
            _process(nchunk - 1, 1)
        plsc.subcore_barrier()

        @pl.when(sid < NS - 1)
        def _drain_body():
            pltpu.sync_copy(
                acc.at[pl.ds(sid * RPT, RPT)],
                out_hbm.at[pl.ds(cid * NN + sid * RPT, RPT)],
            )

        @pl.when(sid == NS - 1)
        def _drain_last():
            pltpu.sync_copy(
                acc.at[pl.ds((NS - 1) * RPT, RPT_LAST)],
                out_hbm.at[pl.ds(cid * NN + (NS - 1) * RPT, RPT_LAST)],
            )

    return _sc_scatter


_sc_scatters = [
    _make_sc_scatter(b, s // NW, 80, w)
    for b, s, w in zip(SC_BASES, SC_SPLITS, W_OFF)
]


def _add_body(a_ref, b_ref, c_ref, d_ref, e_ref, f_ref, o_ref):
    o_ref[...] = ((a_ref[...] + b_ref[...]) + (c_ref[...] + d_ref[...])
                  + (e_ref[...] + f_ref[...]))


_combine_call = pl.pallas_call(
    _add_body,
    grid=(NN // ADD_BLK,),
    in_specs=[pl.BlockSpec((ADD_BLK, D), lambda i: (i, 0))] * 6,
    out_specs=pl.BlockSpec((ADD_BLK, D), lambda i: (i, 0)),
    out_shape=jax.ShapeDtypeStruct((NN, D), jnp.float32),
)


def kernel(msg, x_i, x_j, e_ij, index, num_nodes, W, b):
    w1 = W[:D].T
    w2 = W[D:D + DE].T
    w3 = W[D + DE:].T
    idx = jnp.minimum(index, num_nodes - 1).astype(jnp.int32)
    parts = []
    for gate_call, sc_call in zip(_gate_calls, _sc_scatters):
        gate = gate_call(x_j, e_ij, x_i, w1, w2, w3, b)
        p = sc_call(msg, gate, idx)
        parts.extend([p[:NN], p[NN:]])
    return _combine_call(*parts)


# speedup vs baseline: 1.0608x; 1.0608x over previous
import functools

import jax
import jax.numpy as jnp
from jax import lax
from jax.experimental import pallas as pl
from jax.experimental.pallas import tpu as pltpu
from jax.experimental.pallas import tpu_sc as plsc

E = 320000
NN = 10000
D = 128
DE = 16

NC = 2
NS = 16
NW = NC * NS

GATE_BLK = 8192
SC_SPLITS = (163840, 156160)
SC_BASES = (0, 163840)
GATE_OFF = (0, 20)
GATE_GRID = (20, 20)
W_OFF = (0, 0)

RPT = 624
RPT_LAST = 640
ZR = 80

ADD_BLK = 2000


def _gate_body(xj_ref, ei_ref, xi_ref, w1_ref, w2_ref, w3_ref, b_ref, out_ref):
    dn = (((1,), (1,)), ((), ()))
    s = jax.lax.dot_general(w1_ref[...], xj_ref[...], dn,
                            preferred_element_type=jnp.float32)
    s = s + jax.lax.dot_general(w2_ref[...], ei_ref[...], dn,
                                preferred_element_type=jnp.float32)
    s = s + jax.lax.dot_general(w3_ref[...], xi_ref[...], dn,
                                preferred_element_type=jnp.float32)
    out_ref[...] = jnp.tanh(s + b_ref[0])[0]


def _make_gate_call(off, grid):
    return pl.pallas_call(
        _gate_body,
        grid=(grid,),
        in_specs=[
            pl.BlockSpec((GATE_BLK, D), lambda i: (i + off, 0)),
            pl.BlockSpec((GATE_BLK, DE), lambda i: (i + off, 0)),
            pl.BlockSpec((GATE_BLK, D), lambda i: (i + off, 0)),
            pl.BlockSpec((1, D), lambda i: (0, 0)),
            pl.BlockSpec((1, DE), lambda i: (0, 0)),
            pl.BlockSpec((1, D), lambda i: (0, 0)),
            pl.BlockSpec((1,), lambda i: (0,)),
        ],
        out_specs=pl.BlockSpec((GATE_BLK,), lambda i: (i,)),
        out_shape=jax.ShapeDtypeStruct((grid * GATE_BLK,), jnp.float32),
    )


_gate_calls = [_make_gate_call(o, g) for o, g in zip(GATE_OFF, GATE_GRID)]


_mesh = plsc.VectorSubcoreMesh(core_axis_name="c", subcore_axis_name="s")


def _make_sc_scatter(e_base, epw, CH, w_off):
    nchunk = epw // CH
    assert nchunk * CH == epw and epw % 8 == 0 and CH % 8 == 0 and CH <= 128
    assert CH % 2 == 0 and w_off % 8 == 0

    @functools.partial(
        pl.kernel,
        mesh=_mesh,
        out_type=jax.ShapeDtypeStruct((NC * NN, D), jnp.float32),
        scratch_types=[
            pltpu.VMEM((2, CH, D), jnp.float32),
            pltpu.VMEM((2, CH), jnp.float32),
            pltpu.VMEM((2, CH), jnp.int32),
            pltpu.VMEM((ZR, D), jnp.float32),
            pltpu.VMEM_SHARED((NN, D), jnp.float32),
            pltpu.SemaphoreType.DMA,
            pltpu.SemaphoreType.DMA,
        ],
        compiler_params=pltpu.CompilerParams(needs_layout_passes=False),
    )
    def _sc_scatter(msg_hbm, w_hbm, idx_hbm, out_hbm, msg_v, w_v, idx_v, z_v,
                    acc, isem0, isem1):
        cid = lax.axis_index("c")
        sid = lax.axis_index("s")
        wid = cid * NS + sid
        rbase = w_off + wid * epw
        abase = e_base + wid * epw
        isems = (isem0, isem1)

        def _zrow(r, carry):
            for c in range(D // 16):
                z_v[r, pl.ds(c * 16, 16)] = jnp.zeros((16,), jnp.float32)
            return carry

        lax.fori_loop(0, ZR, _zrow, 0)

        def _zcopy(k, carry):
            pltpu.sync_copy(z_v, acc.at[pl.ds(sid * RPT + k * ZR, ZR)])
            return carry

        lax.fori_loop(0, RPT_LAST // ZR, _zcopy, 0)
        plsc.subcore_barrier()

        def _in_dmas(j, b):
            return (
                pltpu.make_async_copy(
                    msg_hbm.at[pl.ds(abase + j * CH, CH)], msg_v.at[b],
                    isems[b]),
                pltpu.make_async_copy(
                    w_hbm.at[pl.ds(rbase + j * CH, CH)], w_v.at[b],
                    isems[b]),
                pltpu.make_async_copy(
                    idx_hbm.at[pl.ds(abase + j * CH, CH)], idx_v.at[b],
                    isems[b]),
            )

        def _start_in(j, b):
            for d in _in_dmas(j, b):
                d.start()

        def _process(j, b):
            for d in _in_dmas(j, b):
                d.wait()

            def _mrow(r2, inner):
                for u in range(2):
                    r = 2 * r2 + u
                    wb = plsc.load_gather(w_v.at[b],
                                          [jnp.full((16,), r, jnp.int32)])
                    for c in range(D // 16):
                        sl = pl.ds(c * 16, 16)
                        msg_v[b, r, sl] = msg_v[b, r, sl] * wb
                return inner

            lax.fori_loop(0, CH // 2, _mrow, 0)
            pltpu.sync_copy(msg_v.at[b], acc.at[idx_v.at[b]], add=True)

        _start_in(0, 0)

        def _pair(k, carry):
            j0 = 2 * k
            _start_in(j0 + 1, 1)
            _process(j0, 0)
            _start_in(j0 + 2, 0)
            _process(j0 + 1, 1)
            return carry

        if nchunk % 2 == 1:
            lax.fori_loop(0, (nchunk - 1) // 2, _pair, 0)
            _process(nchunk - 1, 0)
        else:
            lax.fori_loop(0, (nchunk - 2) // 2, _pair, 0)
            _start_in(nchunk - 1, 1)
            _process(nchunk - 2, 0)
            _process(nchunk - 1, 1)
        plsc.subcore_barrier()

        @pl.when(sid < NS - 1)
        def _drain_body():
            pltpu.sync_copy(
                acc.at[pl.ds(sid * RPT, RPT)],
                out_hbm.at[pl.ds(cid * NN + sid * RPT, RPT)],
            )

        @pl.when(sid == NS - 1)
        def _drain_last():
            pltpu.sync_copy(
                acc.at[pl.ds((NS - 1) * RPT, RPT_LAST)],
                out_hbm.at[pl.ds(cid * NN + (NS - 1) * RPT, RPT_LAST)],
            )

    return _sc_scatter


_sc_scatters = [
    _make_sc_scatter(b, s // NW, 80, w)
    for b, s, w in zip(SC_BASES, SC_SPLITS, W_OFF)
]


def _add_body(a_ref, b_ref, c_ref, d_ref, o_ref):
    o_ref[...] = (a_ref[...] + b_ref[...]) + (c_ref[...] + d_ref[...])


_combine_call = pl.pallas_call(
    _add_body,
    grid=(NN // ADD_BLK,),
    in_specs=[pl.BlockSpec((ADD_BLK, D), lambda i: (i, 0))] * 4,
    out_specs=pl.BlockSpec((ADD_BLK, D), lambda i: (i, 0)),
    out_shape=jax.ShapeDtypeStruct((NN, D), jnp.float32),
)


def kernel(msg, x_i, x_j, e_ij, index, num_nodes, W, b):
    w1 = W[:D].T
    w2 = W[D:D + DE].T
    w3 = W[D + DE:].T
    idx = jnp.minimum(index, num_nodes - 1).astype(jnp.int32)
    parts = []
    for gate_call, sc_call in zip(_gate_calls, _sc_scatters):
        gate = gate_call(x_j, e_ij, x_i, w1, w2, w3, b)
        p = sc_call(msg, gate, idx)
        parts.extend([p[:NN], p[NN:]])
    return _combine_call(*parts)
